# baseline (speedup 1.0000x reference)
import dataclasses
import functools

import jax
import jax.numpy as jnp
from jax import lax
from jax.experimental import pallas as pl
from jax.experimental.pallas import tpu as pltpu
from jax.experimental.pallas import tpu_sc as plsc

N_C = 10000
N_J = 10000
E = 320000
L = 100000
D_IN = 384
H = 128

NPAD = 10112
RPS = NPAD // 16
ECHUNK = 128
NCH_E = 160
NBI = 32
E_PAD = NCH_E * ECHUNK * 16
LCH_W = 25
L_PAD = LCH_W * ECHUNK * 32
NB = 10
BR = N_C // NB

_HIGH = jax.lax.Precision.HIGHEST


def _dot(a, b):
    return jnp.dot(a, b, preferred_element_type=jnp.float32, precision=_HIGH)


def _enc_body(x_ref, w_ref, b_ref, e_ref, o_ref):
    o_ref[...] = _dot(x_ref[...], w_ref[...]) + b_ref[...] + e_ref[...]


def _encode(x, w, b, emb):
    return pl.pallas_call(
        _enc_body,
        grid=(NB,),
        in_specs=[
            pl.BlockSpec((BR, D_IN), lambda i: (i, 0)),
            pl.BlockSpec((D_IN, H), lambda i: (0, 0)),
            pl.BlockSpec((1, H), lambda i: (0, 0)),
            pl.BlockSpec((BR, H), lambda i: (i, 0)),
        ],
        out_specs=pl.BlockSpec((BR, H), lambda i: (i, 0)),
        out_shape=jax.ShapeDtypeStruct((N_C, H), jnp.float32),
    )(x, w, b.reshape(1, H), emb)


def _layer_body(s_ref, c_ref, x_ref, wl_ref, wr_ref, b_ref, o_ref, *, relu):
    cnt = c_ref[0][:, 0:1]
    m = s_ref[0] * (1.0 / jnp.maximum(cnt, 1.0))
    r = _dot(m, wl_ref[0]) + _dot(x_ref[...], wr_ref[0]) + b_ref[0]
    if relu:
        r = jnp.maximum(r, 0.0)
    o_ref[...] = r


def _layer(sums, cnts, xcj, wl, wr, b, relu):
    return pl.pallas_call(
        functools.partial(_layer_body, relu=relu),
        grid=(2, NB),
        in_specs=[
            pl.BlockSpec((1, BR, H), lambda t, i: (t, i, 0)),
            pl.BlockSpec((1, BR, H), lambda t, i: (t, i, 0)),
            pl.BlockSpec((BR, H), lambda t, i: (t * NB + i, 0)),
            pl.BlockSpec((1, H, H), lambda t, i: (t, 0, 0)),
            pl.BlockSpec((1, H, H), lambda t, i: (t, 0, 0)),
            pl.BlockSpec((1, 1, H), lambda t, i: (t, 0, 0)),
        ],
        out_specs=pl.BlockSpec((BR, H), lambda t, i: (t * NB + i, 0)),
        out_shape=jax.ShapeDtypeStruct((2 * N_C, H), jnp.float32),
    )(sums, cnts, xcj, wl, wr, b.reshape(2, 1, H))


_MESH = plsc.VectorSubcoreMesh(core_axis_name="c", subcore_axis_name="s")

_CP = pltpu.CompilerParams()
if "needs_layout_passes" in pltpu.CompilerParams.__dataclass_fields__:
    _CP = dataclasses.replace(_CP, needs_layout_passes=False)


@functools.partial(
    pl.kernel,
    out_type=jax.ShapeDtypeStruct((2, NPAD, H), jnp.float32),
    mesh=_MESH,
    scratch_types=[
        pltpu.VMEM((1, ECHUNK), jnp.int32),
        pltpu.VMEM((1, ECHUNK), jnp.int32),
        pltpu.VMEM((ECHUNK, H), jnp.float32),
        pltpu.VMEM_SHARED((NPAD, H), jnp.float32),
    ],
)
def _sc_agg(xcj_hbm, src_hbm, dst_hbm, zeros_hbm, sums_hbm,
            srcv, dstv, rows, acc):
    cid = lax.axis_index("c")
    sid = lax.axis_index("s")
    r0 = sid * RPS
    pltpu.sync_copy(zeros_hbm.at[pl.ds(r0, RPS)], acc.at[pl.ds(r0, RPS)])
    plsc.subcore_barrier()

    base = (cid * 16 + sid) * NCH_E

    @pl.loop(0, NCH_E)
    def _edge_chunk(i):
        off = (base + i) * ECHUNK
        pltpu.sync_copy(src_hbm.at[pl.ds(off, ECHUNK)], srcv.at[0])
        pltpu.sync_copy(dst_hbm.at[pl.ds(off, ECHUNK)], dstv.at[0])
        pltpu.sync_copy(xcj_hbm.at[srcv.at[0]], rows)
        pltpu.sync_copy(rows, acc.at[dstv.at[0]], add=True)

    plsc.subcore_barrier()
    pltpu.sync_copy(acc.at[pl.ds(r0, RPS)], sums_hbm.at[cid, pl.ds(r0, RPS)])


@functools.partial(
    pl.kernel,
    out_type=jax.ShapeDtypeStruct((2, NPAD, H), jnp.float32),
    mesh=_MESH,
    scratch_types=[
        pltpu.VMEM((NCH_E, ECHUNK), jnp.int32),
        pltpu.VMEM((ECHUNK, H), jnp.float32),
        pltpu.VMEM_SHARED((NPAD, H), jnp.float32),
    ],
)
def _sc_counts(dst_hbm, zeros_hbm, ones_hbm, cnts_hbm, dstv, onesv, acc):
    cid = lax.axis_index("c")
    sid = lax.axis_index("s")
    r0 = sid * RPS
    pltpu.sync_copy(zeros_hbm.at[pl.ds(r0, RPS)], acc.at[pl.ds(r0, RPS)])
    rowbase = (cid * 16 + sid) * NCH_E
    pltpu.sync_copy(dst_hbm.at[pl.ds(rowbase, NCH_E)], dstv)
    pltpu.sync_copy(ones_hbm, onesv)
    plsc.subcore_barrier()

    @pl.loop(0, NCH_E)
    def _edge_chunk(i):
        pltpu.sync_copy(onesv, acc.at[dstv.at[i]], add=True)

    plsc.subcore_barrier()
    pltpu.sync_copy(acc.at[pl.ds(r0, RPS)], cnts_hbm.at[cid, pl.ds(r0, RPS)])


@functools.partial(
    pl.kernel,
    out_type=jax.ShapeDtypeStruct((L_PAD,), jnp.float32),
    mesh=_MESH,
    scratch_types=[
        pltpu.VMEM((1, ECHUNK), jnp.int32),
        pltpu.VMEM((1, ECHUNK), jnp.int32),
        pltpu.VMEM((ECHUNK, H), jnp.float32),
        pltpu.VMEM((ECHUNK, H), jnp.float32),
        pltpu.VMEM((1, ECHUNK), jnp.float32),
    ],
    compiler_params=_CP,
)
def _sc_classify(xcj_hbm, ia_hbm, ib_hbm, out_hbm, iav, ibv, ra, rb, outv):
    cid = lax.axis_index("c")
    sid = lax.axis_index("s")
    base = (cid * 16 + sid) * LCH_W

    @pl.loop(0, LCH_W)
    def _label_chunk(c):
        goff = (base + c) * ECHUNK
        pltpu.sync_copy(ia_hbm.at[pl.ds(goff, ECHUNK)], iav.at[0])
        pltpu.sync_copy(ib_hbm.at[pl.ds(goff, ECHUNK)], ibv.at[0])
        pltpu.sync_copy(xcj_hbm.at[iav.at[0]], ra)
        pltpu.sync_copy(xcj_hbm.at[ibv.at[0]], rb)

        @pl.loop(0, ECHUNK // 16)
        def _group(grp):
            ridx = grp * 16 + lax.iota(jnp.int32, 16)

            def h_body(h, acc):
                ch = jnp.full((16,), h, jnp.int32)
                va = plsc.load_gather(ra, [ridx, ch])
                vb = plsc.load_gather(rb, [ridx, ch])
                return acc + va * vb

            acc = lax.fori_loop(0, H, h_body, jnp.zeros((16,), jnp.float32),
                                unroll=8)
            outv[0, pl.ds(grp * 16, 16)] = acc

        pltpu.sync_copy(outv.at[0], out_hbm.at[pl.ds(goff, ECHUNK)])


def kernel(x_user, x_item, node_id_user, node_id_item, edge_index,
           edge_label_index, W_user_lin, b_user_lin, W_job_lin, b_job_lin,
           user_emb, job_emb,
           Wl0_c2j, Wr0_c2j, b0_c2j, Wl0_j2c, Wr0_j2c, b0_j2c,
           Wl1_c2j, Wr1_c2j, b1_c2j, Wl1_j2c, Wr1_j2c, b1_j2c,
           Wl2_c2j, Wr2_c2j, b2_c2j, Wl2_j2c, Wr2_j2c, b2_j2c):
    i32 = jnp.int32
    ei0 = edge_index[0]
    ei1 = edge_index[1]
    epad = E_PAD - E
    padz = jnp.zeros((epad,), i32)
    padg = jnp.full((epad,), N_C, i32)
    src_flat = jnp.concatenate([ei1 + N_C, padz, ei0, padz])
    dst_flat = jnp.concatenate([ei0, padg, ei1, padg])
    dst2d = dst_flat.reshape(-1, ECHUNK)

    lpad = L_PAD - L
    lpz = jnp.zeros((lpad,), i32)
    ia = jnp.concatenate([edge_label_index[0], lpz])
    ib = jnp.concatenate([edge_label_index[1] + N_C, lpz])

    zeros = jnp.zeros((NPAD, H), jnp.float32)
    ones = jnp.ones((ECHUNK, H), jnp.float32)

    x_c = _encode(x_user, W_user_lin, b_user_lin, user_emb)
    x_j = _encode(x_item, W_job_lin, b_job_lin, job_emb)
    xcj = jnp.concatenate([x_c, x_j], axis=0)

    wl = (jnp.stack([Wl0_j2c, Wl0_c2j]), jnp.stack([Wl1_j2c, Wl1_c2j]),
          jnp.stack([Wl2_j2c, Wl2_c2j]))
    wr = (jnp.stack([Wr0_j2c, Wr0_c2j]), jnp.stack([Wr1_j2c, Wr1_c2j]),
          jnp.stack([Wr2_j2c, Wr2_c2j]))
    bb = (jnp.stack([b0_j2c, b0_c2j]), jnp.stack([b1_j2c, b1_c2j]),
          jnp.stack([b2_j2c, b2_c2j]))

    cnts = _sc_counts(dst2d, zeros, ones)
    for l in range(3):
        sums = _sc_agg(xcj, src_flat, dst_flat, zeros)
        xcj = _layer(sums, cnts, xcj, wl[l], wr[l], bb[l], relu=(l == 0))

    pred = _sc_classify(xcj, ia, ib)
    return pred[:L]


# speedup vs baseline: 1.5990x; 1.5990x over previous
import dataclasses
import functools

import jax
import jax.numpy as jnp
from jax import lax
from jax.experimental import pallas as pl
from jax.experimental.pallas import tpu as pltpu
from jax.experimental.pallas import tpu_sc as plsc

N_C = 10000
N_J = 10000
E = 320000
L = 100000
D_IN = 384
H = 128

NPAD = 10112
RPS = NPAD // 16
ECHUNK = 128
NCH_E = 157
NCH_C = 160
E_PAD = NCH_E * ECHUNK * 16
LCH_W = 25
L_PAD = LCH_W * ECHUNK * 32
NB = 10
BR = N_C // NB

_HIGH = jax.lax.Precision.HIGHEST


def _dot(a, b):
    return jnp.dot(a, b, preferred_element_type=jnp.float32, precision=_HIGH)


def _enc_body(x_ref, w_ref, b_ref, e_ref, o_ref):
    o_ref[...] = _dot(x_ref[...], w_ref[...]) + b_ref[...] + e_ref[...]


def _encode(x, w, b, emb):
    return pl.pallas_call(
        _enc_body,
        grid=(NB,),
        in_specs=[
            pl.BlockSpec((BR, D_IN), lambda i: (i, 0)),
            pl.BlockSpec((D_IN, H), lambda i: (0, 0)),
            pl.BlockSpec((1, H), lambda i: (0, 0)),
            pl.BlockSpec((BR, H), lambda i: (i, 0)),
        ],
        out_specs=pl.BlockSpec((BR, H), lambda i: (i, 0)),
        out_shape=jax.ShapeDtypeStruct((N_C, H), jnp.float32),
    )(x, w, b.reshape(1, H), emb)


def _layer_body(s_ref, c_ref, x_ref, wl_ref, wr_ref, b_ref, o_ref, *, relu):
    cnt = c_ref[0][:, 0:1]
    m = s_ref[0] * (1.0 / jnp.maximum(cnt, 1.0))
    r = _dot(m, wl_ref[0]) + _dot(x_ref[...], wr_ref[0]) + b_ref[0]
    if relu:
        r = jnp.maximum(r, 0.0)
    o_ref[...] = r


def _layer(sums, cnts, xcj, wl, wr, b, relu):
    return pl.pallas_call(
        functools.partial(_layer_body, relu=relu),
        grid=(2, NB),
        in_specs=[
            pl.BlockSpec((1, BR, H), lambda t, i: (t, i, 0)),
            pl.BlockSpec((1, BR, H), lambda t, i: (t, i, 0)),
            pl.BlockSpec((BR, H), lambda t, i: (t * NB + i, 0)),
            pl.BlockSpec((1, H, H), lambda t, i: (t, 0, 0)),
            pl.BlockSpec((1, H, H), lambda t, i: (t, 0, 0)),
            pl.BlockSpec((1, 1, H), lambda t, i: (t, 0, 0)),
        ],
        out_specs=pl.BlockSpec((BR, H), lambda t, i: (t * NB + i, 0)),
        out_shape=jax.ShapeDtypeStruct((2 * N_C, H), jnp.float32),
    )(sums, cnts, xcj, wl, wr, b.reshape(2, 1, H))


_MESH = plsc.VectorSubcoreMesh(core_axis_name="c", subcore_axis_name="s")

_CP = pltpu.CompilerParams()
if "needs_layout_passes" in pltpu.CompilerParams.__dataclass_fields__:
    _CP = dataclasses.replace(_CP, needs_layout_passes=False)


@functools.partial(
    pl.kernel,
    out_type=jax.ShapeDtypeStruct((2, NPAD, H), jnp.float32),
    mesh=_MESH,
    scratch_types=[
        pltpu.VMEM((1, ECHUNK), jnp.int32),
        pltpu.VMEM((1, ECHUNK), jnp.int32),
        pltpu.VMEM((ECHUNK, H), jnp.float32),
        pltpu.VMEM_SHARED((NPAD, H), jnp.float32),
    ],
)
def _sc_agg(xcj_hbm, src_hbm, dst_hbm, zeros_hbm, sums_hbm,
            srcv, dstv, rows, acc):
    cid = lax.axis_index("c")
    sid = lax.axis_index("s")
    r0 = sid * RPS
    pltpu.sync_copy(zeros_hbm.at[pl.ds(r0, RPS)], acc.at[pl.ds(r0, RPS)])
    plsc.subcore_barrier()

    base = (cid * 16 + sid) * NCH_E

    @pl.loop(0, NCH_E)
    def _edge_chunk(i):
        off = (base + i) * ECHUNK
        pltpu.sync_copy(src_hbm.at[pl.ds(off, ECHUNK)], srcv.at[0])
        pltpu.sync_copy(dst_hbm.at[pl.ds(off, ECHUNK)], dstv.at[0])
        pltpu.sync_copy(xcj_hbm.at[srcv.at[0]], rows)
        pltpu.sync_copy(rows, acc.at[dstv.at[0]], add=True)

    plsc.subcore_barrier()
    pltpu.sync_copy(acc.at[pl.ds(r0, RPS)], sums_hbm.at[cid, pl.ds(r0, RPS)])


@functools.partial(
    pl.kernel,
    out_type=jax.ShapeDtypeStruct((2, NPAD, H), jnp.float32),
    mesh=_MESH,
    scratch_types=[
        pltpu.VMEM((NCH_C, ECHUNK), jnp.int32),
        pltpu.VMEM((ECHUNK, H), jnp.float32),
        pltpu.VMEM_SHARED((NPAD, H), jnp.float32),
    ],
)
def _sc_counts(dst_hbm, zeros_hbm, ones_hbm, cnts_hbm, dstv, onesv, acc):
    cid = lax.axis_index("c")
    sid = lax.axis_index("s")
    r0 = sid * RPS
    pltpu.sync_copy(zeros_hbm.at[pl.ds(r0, RPS)], acc.at[pl.ds(r0, RPS)])
    rowbase = (cid * 16 + sid) * NCH_C
    pltpu.sync_copy(dst_hbm.at[pl.ds(rowbase, NCH_C)], dstv)
    pltpu.sync_copy(ones_hbm, onesv)
    plsc.subcore_barrier()

    @pl.loop(0, NCH_C)
    def _edge_chunk(i):
        pltpu.sync_copy(onesv, acc.at[dstv.at[i]], add=True)

    plsc.subcore_barrier()
    pltpu.sync_copy(acc.at[pl.ds(r0, RPS)], cnts_hbm.at[cid, pl.ds(r0, RPS)])


@functools.partial(
    pl.kernel,
    out_type=jax.ShapeDtypeStruct((L_PAD,), jnp.float32),
    mesh=_MESH,
    scratch_types=[
        pltpu.VMEM((1, ECHUNK), jnp.int32),
        pltpu.VMEM((1, ECHUNK), jnp.int32),
        pltpu.VMEM((ECHUNK, H), jnp.float32),
        pltpu.VMEM((ECHUNK, H), jnp.float32),
        pltpu.VMEM((1, ECHUNK), jnp.float32),
    ],
    compiler_params=_CP,
)
def _sc_classify(xcj_hbm, ia_hbm, ib_hbm, out_hbm, iav, ibv, ra, rb, outv):
    cid = lax.axis_index("c")
    sid = lax.axis_index("s")
    base = (cid * 16 + sid) * LCH_W

    @pl.loop(0, LCH_W)
    def _label_chunk(c):
        goff = (base + c) * ECHUNK
        pltpu.sync_copy(ia_hbm.at[pl.ds(goff, ECHUNK)], iav.at[0])
        pltpu.sync_copy(ib_hbm.at[pl.ds(goff, ECHUNK)], ibv.at[0])
        pltpu.sync_copy(xcj_hbm.at[iav.at[0]], ra)
        pltpu.sync_copy(xcj_hbm.at[ibv.at[0]], rb)

        @pl.loop(0, ECHUNK // 16)
        def _group(grp):
            ridx = grp * 16 + lax.iota(jnp.int32, 16)

            def h_body(h, acc):
                ch = jnp.full((16,), h, jnp.int32)
                va = plsc.load_gather(ra, [ridx, ch])
                vb = plsc.load_gather(rb, [ridx, ch])
                return acc + va * vb

            acc = lax.fori_loop(0, H, h_body, jnp.zeros((16,), jnp.float32),
                                unroll=8)
            outv[0, pl.ds(grp * 16, 16)] = acc

        pltpu.sync_copy(outv.at[0], out_hbm.at[pl.ds(goff, ECHUNK)])


def kernel(x_user, x_item, node_id_user, node_id_item, edge_index,
           edge_label_index, W_user_lin, b_user_lin, W_job_lin, b_job_lin,
           user_emb, job_emb,
           Wl0_c2j, Wr0_c2j, b0_c2j, Wl0_j2c, Wr0_j2c, b0_j2c,
           Wl1_c2j, Wr1_c2j, b1_c2j, Wl1_j2c, Wr1_j2c, b1_j2c,
           Wl2_c2j, Wr2_c2j, b2_c2j, Wl2_j2c, Wr2_j2c, b2_j2c):
    i32 = jnp.int32
    ei0 = edge_index[0]
    ei1 = edge_index[1]
    epad = E_PAD - E
    padz = jnp.remainder(jnp.arange(epad, dtype=i32), 2048)
    padg = N_C + jnp.remainder(jnp.arange(epad, dtype=i32), NPAD - N_C)
    src_flat = jnp.concatenate([ei1 + N_C, padz, ei0, padz])
    dst_flat = jnp.concatenate([ei0, padg, ei1, padg])
    epad_c = NCH_C * ECHUNK * 16 - E
    padg_c = N_C + jnp.remainder(jnp.arange(epad_c, dtype=i32), NPAD - N_C)
    dst2d = jnp.concatenate([ei0, padg_c, ei1, padg_c]).reshape(-1, ECHUNK)

    lpad = L_PAD - L
    lpz = jnp.zeros((lpad,), i32)
    ia = jnp.concatenate([edge_label_index[0], lpz])
    ib = jnp.concatenate([edge_label_index[1] + N_C, lpz])

    zeros = jnp.zeros((NPAD, H), jnp.float32)
    ones = jnp.ones((ECHUNK, H), jnp.float32)

    x_c = _encode(x_user, W_user_lin, b_user_lin, user_emb)
    x_j = _encode(x_item, W_job_lin, b_job_lin, job_emb)
    xcj = jnp.concatenate([x_c, x_j], axis=0)

    wl = (jnp.stack([Wl0_j2c, Wl0_c2j]), jnp.stack([Wl1_j2c, Wl1_c2j]),
          jnp.stack([Wl2_j2c, Wl2_c2j]))
    wr = (jnp.stack([Wr0_j2c, Wr0_c2j]), jnp.stack([Wr1_j2c, Wr1_c2j]),
          jnp.stack([Wr2_j2c, Wr2_c2j]))
    bb = (jnp.stack([b0_j2c, b0_c2j]), jnp.stack([b1_j2c, b1_c2j]),
          jnp.stack([b2_j2c, b2_c2j]))

    cnts = _sc_counts(dst2d, zeros, ones)
    for l in range(3):
        sums = _sc_agg(xcj, src_flat, dst_flat, zeros)
        xcj = _layer(sums, cnts, xcj, wl[l], wr[l], bb[l], relu=(l == 0))

    pred = _sc_classify(xcj, ia, ib)
    return pred[:L]


# speedup vs baseline: 1.8413x; 1.1515x over previous
import dataclasses
import functools

import jax
import jax.numpy as jnp
from jax import lax
from jax.experimental import pallas as pl
from jax.experimental.pallas import tpu as pltpu
from jax.experimental.pallas import tpu_sc as plsc

N_C = 10000
N_J = 10000
E = 320000
L = 100000
D_IN = 384
H = 128

NPAD = 10112
RPS = NPAD // 16
ECHUNK = 128
NCH_E = 157
NCH_C = 160
E_PAD = NCH_E * ECHUNK * 16
LCH_W = 25
L_PAD = LCH_W * ECHUNK * 32
NB = 10
BR = N_C // NB

_HIGH = jax.lax.Precision.HIGHEST


def _dot(a, b):
    return jnp.dot(a, b, preferred_element_type=jnp.float32, precision=_HIGH)


def _enc_body(x_ref, w_ref, b_ref, e_ref, o_ref):
    o_ref[...] = _dot(x_ref[...], w_ref[...]) + b_ref[...] + e_ref[...]


def _encode(x, w, b, emb):
    return pl.pallas_call(
        _enc_body,
        grid=(NB,),
        in_specs=[
            pl.BlockSpec((BR, D_IN), lambda i: (i, 0)),
            pl.BlockSpec((D_IN, H), lambda i: (0, 0)),
            pl.BlockSpec((1, H), lambda i: (0, 0)),
            pl.BlockSpec((BR, H), lambda i: (i, 0)),
        ],
        out_specs=pl.BlockSpec((BR, H), lambda i: (i, 0)),
        out_shape=jax.ShapeDtypeStruct((N_C, H), jnp.float32),
    )(x, w, b.reshape(1, H), emb)


def _layer_body(s_ref, c_ref, x_ref, wl_ref, wr_ref, b_ref, o_ref, *, relu):
    cnt = c_ref[0][:, 0:1]
    m = s_ref[0] * (1.0 / jnp.maximum(cnt, 1.0))
    r = _dot(m, wl_ref[0]) + _dot(x_ref[...], wr_ref[0]) + b_ref[0]
    if relu:
        r = jnp.maximum(r, 0.0)
    o_ref[...] = r


def _layer(sums, cnts, xcj, wl, wr, b, relu):
    return pl.pallas_call(
        functools.partial(_layer_body, relu=relu),
        grid=(2, NB),
        in_specs=[
            pl.BlockSpec((1, BR, H), lambda t, i: (t, i, 0)),
            pl.BlockSpec((1, BR, H), lambda t, i: (t, i, 0)),
            pl.BlockSpec((BR, H), lambda t, i: (t * NB + i, 0)),
            pl.BlockSpec((1, H, H), lambda t, i: (t, 0, 0)),
            pl.BlockSpec((1, H, H), lambda t, i: (t, 0, 0)),
            pl.BlockSpec((1, 1, H), lambda t, i: (t, 0, 0)),
        ],
        out_specs=pl.BlockSpec((BR, H), lambda t, i: (t * NB + i, 0)),
        out_shape=jax.ShapeDtypeStruct((2 * N_C, H), jnp.float32),
    )(sums, cnts, xcj, wl, wr, b.reshape(2, 1, H))


_MESH = plsc.VectorSubcoreMesh(core_axis_name="c", subcore_axis_name="s")

_CP = pltpu.CompilerParams()
if "needs_layout_passes" in pltpu.CompilerParams.__dataclass_fields__:
    _CP = dataclasses.replace(_CP, needs_layout_passes=False)


@functools.partial(
    pl.kernel,
    out_type=jax.ShapeDtypeStruct((2, NPAD, H), jnp.float32),
    mesh=_MESH,
    scratch_types=[
        pltpu.VMEM((1, ECHUNK), jnp.int32),
        pltpu.VMEM((1, ECHUNK), jnp.int32),
        pltpu.VMEM((ECHUNK, H), jnp.float32),
        pltpu.VMEM_SHARED((NPAD, H), jnp.float32),
    ],
)
def _sc_agg(xcj_hbm, src_hbm, dst_hbm, zeros_hbm, sums_hbm,
            srcv, dstv, rows, acc):
    cid = lax.axis_index("c")
    sid = lax.axis_index("s")
    r0 = sid * RPS
    pltpu.sync_copy(zeros_hbm.at[pl.ds(r0, RPS)], acc.at[pl.ds(r0, RPS)])
    plsc.subcore_barrier()

    base = (cid * 16 + sid) * NCH_E

    @pl.loop(0, NCH_E)
    def _edge_chunk(i):
        off = (base + i) * ECHUNK
        pltpu.sync_copy(src_hbm.at[pl.ds(off, ECHUNK)], srcv.at[0])
        pltpu.sync_copy(dst_hbm.at[pl.ds(off, ECHUNK)], dstv.at[0])
        pltpu.sync_copy(xcj_hbm.at[srcv.at[0]], rows)
        pltpu.sync_copy(rows, acc.at[dstv.at[0]], add=True)

    plsc.subcore_barrier()
    pltpu.sync_copy(acc.at[pl.ds(r0, RPS)], sums_hbm.at[cid, pl.ds(r0, RPS)])


@functools.partial(
    pl.kernel,
    out_type=jax.ShapeDtypeStruct((2, NPAD, H), jnp.float32),
    mesh=_MESH,
    scratch_types=[
        pltpu.VMEM((NCH_C, ECHUNK), jnp.int32),
        pltpu.VMEM((ECHUNK, H), jnp.float32),
        pltpu.VMEM_SHARED((NPAD, H), jnp.float32),
    ],
)
def _sc_counts(dst_hbm, zeros_hbm, ones_hbm, cnts_hbm, dstv, onesv, acc):
    cid = lax.axis_index("c")
    sid = lax.axis_index("s")
    r0 = sid * RPS
    pltpu.sync_copy(zeros_hbm.at[pl.ds(r0, RPS)], acc.at[pl.ds(r0, RPS)])
    rowbase = (cid * 16 + sid) * NCH_C
    pltpu.sync_copy(dst_hbm.at[pl.ds(rowbase, NCH_C)], dstv)
    pltpu.sync_copy(ones_hbm, onesv)
    plsc.subcore_barrier()

    @pl.loop(0, NCH_C)
    def _edge_chunk(i):
        pltpu.sync_copy(onesv, acc.at[dstv.at[i]], add=True)

    plsc.subcore_barrier()
    pltpu.sync_copy(acc.at[pl.ds(r0, RPS)], cnts_hbm.at[cid, pl.ds(r0, RPS)])


@functools.partial(
    pl.kernel,
    out_type=jax.ShapeDtypeStruct((L_PAD,), jnp.float32),
    mesh=_MESH,
    scratch_types=[
        pltpu.VMEM((1, ECHUNK), jnp.int32),
        pltpu.VMEM((1, ECHUNK), jnp.int32),
        pltpu.VMEM((ECHUNK, H), jnp.float32),
        pltpu.VMEM((ECHUNK, H), jnp.float32),
        pltpu.VMEM((1, ECHUNK), jnp.float32),
        pltpu.VMEM((16, 16), jnp.float32),
    ],
    compiler_params=_CP,
)
def _sc_classify(xcj_hbm, ia_hbm, ib_hbm, out_hbm, iav, ibv, ra, rb, outv,
                 pbuf):
    cid = lax.axis_index("c")
    sid = lax.axis_index("s")
    base = (cid * 16 + sid) * LCH_W

    @pl.loop(0, LCH_W)
    def _label_chunk(c):
        goff = (base + c) * ECHUNK
        pltpu.sync_copy(ia_hbm.at[pl.ds(goff, ECHUNK)], iav.at[0])
        pltpu.sync_copy(ib_hbm.at[pl.ds(goff, ECHUNK)], ibv.at[0])
        pltpu.sync_copy(xcj_hbm.at[iav.at[0]], ra)
        pltpu.sync_copy(xcj_hbm.at[ibv.at[0]], rb)

        @pl.loop(0, ECHUNK // 16)
        def _group(grp):
            @pl.loop(0, 16)
            def _edge(j):
                e = grp * 16 + j
                p = ra[e, pl.ds(0, 16)] * rb[e, pl.ds(0, 16)]
                for k in range(1, H // 16):
                    p = p + ra[e, pl.ds(k * 16, 16)] * rb[e, pl.ds(k * 16, 16)]
                pbuf[j, :] = p

            rid = lax.iota(jnp.int32, 16)
            acc = jnp.zeros((16,), jnp.float32)
            for c in range(16):
                acc = acc + plsc.load_gather(
                    pbuf, [rid, jnp.full((16,), c, jnp.int32)])
            outv[0, pl.ds(grp * 16, 16)] = acc

        pltpu.sync_copy(outv.at[0], out_hbm.at[pl.ds(goff, ECHUNK)])


def kernel(x_user, x_item, node_id_user, node_id_item, edge_index,
           edge_label_index, W_user_lin, b_user_lin, W_job_lin, b_job_lin,
           user_emb, job_emb,
           Wl0_c2j, Wr0_c2j, b0_c2j, Wl0_j2c, Wr0_j2c, b0_j2c,
           Wl1_c2j, Wr1_c2j, b1_c2j, Wl1_j2c, Wr1_j2c, b1_j2c,
           Wl2_c2j, Wr2_c2j, b2_c2j, Wl2_j2c, Wr2_j2c, b2_j2c):
    i32 = jnp.int32
    ei0 = edge_index[0]
    ei1 = edge_index[1]
    epad = E_PAD - E
    padz = jnp.remainder(jnp.arange(epad, dtype=i32), 2048)
    padg = N_C + jnp.remainder(jnp.arange(epad, dtype=i32), NPAD - N_C)
    src_flat = jnp.concatenate([ei1 + N_C, padz, ei0, padz])
    dst_flat = jnp.concatenate([ei0, padg, ei1, padg])
    epad_c = NCH_C * ECHUNK * 16 - E
    padg_c = N_C + jnp.remainder(jnp.arange(epad_c, dtype=i32), NPAD - N_C)
    dst2d = jnp.concatenate([ei0, padg_c, ei1, padg_c]).reshape(-1, ECHUNK)

    lpad = L_PAD - L
    lpz = jnp.zeros((lpad,), i32)
    ia = jnp.concatenate([edge_label_index[0], lpz])
    ib = jnp.concatenate([edge_label_index[1] + N_C, lpz])

    zeros = jnp.zeros((NPAD, H), jnp.float32)
    ones = jnp.ones((ECHUNK, H), jnp.float32)

    x_c = _encode(x_user, W_user_lin, b_user_lin, user_emb)
    x_j = _encode(x_item, W_job_lin, b_job_lin, job_emb)
    xcj = jnp.concatenate([x_c, x_j], axis=0)

    wl = (jnp.stack([Wl0_j2c, Wl0_c2j]), jnp.stack([Wl1_j2c, Wl1_c2j]),
          jnp.stack([Wl2_j2c, Wl2_c2j]))
    wr = (jnp.stack([Wr0_j2c, Wr0_c2j]), jnp.stack([Wr1_j2c, Wr1_c2j]),
          jnp.stack([Wr2_j2c, Wr2_c2j]))
    bb = (jnp.stack([b0_j2c, b0_c2j]), jnp.stack([b1_j2c, b1_c2j]),
          jnp.stack([b2_j2c, b2_c2j]))

    cnts = _sc_counts(dst2d, zeros, ones)
    for l in range(3):
        sums = _sc_agg(xcj, src_flat, dst_flat, zeros)
        xcj = _layer(sums, cnts, xcj, wl[l], wr[l], bb[l], relu=(l == 0))

    pred = _sc_classify(xcj, ia, ib)
    return pred[:L]


# speedup vs baseline: 2.0388x; 1.1073x over previous
import dataclasses
import functools

import jax
import jax.numpy as jnp
from jax import lax
from jax.experimental import pallas as pl
from jax.experimental.pallas import tpu as pltpu
from jax.experimental.pallas import tpu_sc as plsc

N_C = 10000
N_J = 10000
E = 320000
L = 100000
D_IN = 384
H = 128

NPAD = 10112
RPS = NPAD // 16
ECHUNK = 128
NCH_E = 157
NCH_C = 160
E_PAD = NCH_E * ECHUNK * 16
LCH_W = 25
L_PAD = LCH_W * ECHUNK * 32
NB = 10
BR = N_C // NB

_HIGH = jax.lax.Precision.HIGHEST


def _dot(a, b):
    return jnp.dot(a, b, preferred_element_type=jnp.float32, precision=_HIGH)


def _enc_body(x_ref, w_ref, b_ref, e_ref, o_ref):
    o_ref[...] = _dot(x_ref[...], w_ref[...]) + b_ref[...] + e_ref[...]


def _encode(x, w, b, emb):
    return pl.pallas_call(
        _enc_body,
        grid=(NB,),
        in_specs=[
            pl.BlockSpec((BR, D_IN), lambda i: (i, 0)),
            pl.BlockSpec((D_IN, H), lambda i: (0, 0)),
            pl.BlockSpec((1, H), lambda i: (0, 0)),
            pl.BlockSpec((BR, H), lambda i: (i, 0)),
        ],
        out_specs=pl.BlockSpec((BR, H), lambda i: (i, 0)),
        out_shape=jax.ShapeDtypeStruct((N_C, H), jnp.float32),
    )(x, w, b.reshape(1, H), emb)


def _layer_body(s_ref, c_ref, x_ref, wl_ref, wr_ref, b_ref, o_ref, *, relu):
    cnt = c_ref[0][:, 0:1]
    m = s_ref[0] * (1.0 / jnp.maximum(cnt, 1.0))
    r = _dot(m, wl_ref[0]) + _dot(x_ref[...], wr_ref[0]) + b_ref[0]
    if relu:
        r = jnp.maximum(r, 0.0)
    o_ref[...] = r


def _layer(sums, cnts, xcj, wl, wr, b, relu):
    return pl.pallas_call(
        functools.partial(_layer_body, relu=relu),
        grid=(2, NB),
        in_specs=[
            pl.BlockSpec((1, BR, H), lambda t, i: (t, i, 0)),
            pl.BlockSpec((1, BR, H), lambda t, i: (t, i, 0)),
            pl.BlockSpec((BR, H), lambda t, i: (t * NB + i, 0)),
            pl.BlockSpec((1, H, H), lambda t, i: (t, 0, 0)),
            pl.BlockSpec((1, H, H), lambda t, i: (t, 0, 0)),
            pl.BlockSpec((1, 1, H), lambda t, i: (t, 0, 0)),
        ],
        out_specs=pl.BlockSpec((BR, H), lambda t, i: (t * NB + i, 0)),
        out_shape=jax.ShapeDtypeStruct((2 * N_C, H), jnp.float32),
    )(sums, cnts, xcj, wl, wr, b.reshape(2, 1, H))


_MESH = plsc.VectorSubcoreMesh(core_axis_name="c", subcore_axis_name="s")

_CP = pltpu.CompilerParams()
if "needs_layout_passes" in pltpu.CompilerParams.__dataclass_fields__:
    _CP = dataclasses.replace(_CP, needs_layout_passes=False)


@functools.partial(
    pl.kernel,
    out_type=jax.ShapeDtypeStruct((2, NPAD, H), jnp.float32),
    mesh=_MESH,
    scratch_types=[
        pltpu.VMEM((2, ECHUNK), jnp.int32),
        pltpu.VMEM((ECHUNK, H), jnp.float32),
        pltpu.VMEM_SHARED((NPAD, H), jnp.float32),
    ],
)
def _sc_agg(xcj_hbm, sd_hbm, zeros_hbm, sums_hbm, sdv, rows, acc):
    cid = lax.axis_index("c")
    sid = lax.axis_index("s")
    r0 = sid * RPS
    pltpu.sync_copy(zeros_hbm.at[pl.ds(r0, RPS)], acc.at[pl.ds(r0, RPS)])
    plsc.subcore_barrier()

    base = (cid * 16 + sid) * NCH_E

    @pl.loop(0, NCH_E)
    def _edge_chunk(i):
        pltpu.sync_copy(sd_hbm.at[base + i], sdv)
        pltpu.sync_copy(xcj_hbm.at[sdv.at[0]], rows)
        pltpu.sync_copy(rows, acc.at[sdv.at[1]], add=True)

    plsc.subcore_barrier()
    pltpu.sync_copy(acc.at[pl.ds(r0, RPS)], sums_hbm.at[cid, pl.ds(r0, RPS)])


@functools.partial(
    pl.kernel,
    out_type=jax.ShapeDtypeStruct((2, NPAD, H), jnp.float32),
    mesh=_MESH,
    scratch_types=[
        pltpu.VMEM((NCH_C, ECHUNK), jnp.int32),
        pltpu.VMEM((ECHUNK, H), jnp.float32),
        pltpu.VMEM_SHARED((NPAD, H), jnp.float32),
    ],
)
def _sc_counts(dst_hbm, zeros_hbm, ones_hbm, cnts_hbm, dstv, onesv, acc):
    cid = lax.axis_index("c")
    sid = lax.axis_index("s")
    r0 = sid * RPS
    pltpu.sync_copy(zeros_hbm.at[pl.ds(r0, RPS)], acc.at[pl.ds(r0, RPS)])
    rowbase = (cid * 16 + sid) * NCH_C
    pltpu.sync_copy(dst_hbm.at[pl.ds(rowbase, NCH_C)], dstv)
    pltpu.sync_copy(ones_hbm, onesv)
    plsc.subcore_barrier()

    @pl.loop(0, NCH_C)
    def _edge_chunk(i):
        pltpu.sync_copy(onesv, acc.at[dstv.at[i]], add=True)

    plsc.subcore_barrier()
    pltpu.sync_copy(acc.at[pl.ds(r0, RPS)], cnts_hbm.at[cid, pl.ds(r0, RPS)])


@functools.partial(
    pl.kernel,
    out_type=jax.ShapeDtypeStruct((L_PAD,), jnp.float32),
    mesh=_MESH,
    scratch_types=[
        pltpu.VMEM((1, ECHUNK), jnp.int32),
        pltpu.VMEM((1, ECHUNK), jnp.int32),
        pltpu.VMEM((ECHUNK, H), jnp.float32),
        pltpu.VMEM((ECHUNK, H), jnp.float32),
        pltpu.VMEM((1, ECHUNK), jnp.float32),
        pltpu.VMEM((16, 16), jnp.float32),
    ],
    compiler_params=_CP,
)
def _sc_classify(xcj_hbm, ia_hbm, ib_hbm, out_hbm, iav, ibv, ra, rb, outv,
                 pbuf):
    cid = lax.axis_index("c")
    sid = lax.axis_index("s")
    base = (cid * 16 + sid) * LCH_W

    @pl.loop(0, LCH_W)
    def _label_chunk(c):
        goff = (base + c) * ECHUNK
        pltpu.sync_copy(ia_hbm.at[pl.ds(goff, ECHUNK)], iav.at[0])
        pltpu.sync_copy(ib_hbm.at[pl.ds(goff, ECHUNK)], ibv.at[0])
        pltpu.sync_copy(xcj_hbm.at[iav.at[0]], ra)
        pltpu.sync_copy(xcj_hbm.at[ibv.at[0]], rb)

        @pl.loop(0, ECHUNK // 16)
        def _group(grp):
            @pl.loop(0, 16)
            def _edge(j):
                e = grp * 16 + j
                p = ra[e, pl.ds(0, 16)] * rb[e, pl.ds(0, 16)]
                for k in range(1, H // 16):
                    p = p + ra[e, pl.ds(k * 16, 16)] * rb[e, pl.ds(k * 16, 16)]
                pbuf[j, :] = p

            rid = lax.iota(jnp.int32, 16)
            acc = jnp.zeros((16,), jnp.float32)
            for c in range(16):
                acc = acc + plsc.load_gather(
                    pbuf, [rid, jnp.full((16,), c, jnp.int32)])
            outv[0, pl.ds(grp * 16, 16)] = acc

        pltpu.sync_copy(outv.at[0], out_hbm.at[pl.ds(goff, ECHUNK)])


def kernel(x_user, x_item, node_id_user, node_id_item, edge_index,
           edge_label_index, W_user_lin, b_user_lin, W_job_lin, b_job_lin,
           user_emb, job_emb,
           Wl0_c2j, Wr0_c2j, b0_c2j, Wl0_j2c, Wr0_j2c, b0_j2c,
           Wl1_c2j, Wr1_c2j, b1_c2j, Wl1_j2c, Wr1_j2c, b1_j2c,
           Wl2_c2j, Wr2_c2j, b2_c2j, Wl2_j2c, Wr2_j2c, b2_j2c):
    i32 = jnp.int32
    ei0 = edge_index[0]
    ei1 = edge_index[1]
    epad = E_PAD - E
    padz = jnp.remainder(jnp.arange(epad, dtype=i32), 2048)
    padg = N_C + jnp.remainder(jnp.arange(epad, dtype=i32), NPAD - N_C)
    src_flat = jnp.concatenate([ei1 + N_C, padz, ei0, padz])
    dst_flat = jnp.concatenate([ei0, padg, ei1, padg])
    sd3 = jnp.stack([src_flat.reshape(-1, ECHUNK),
                     dst_flat.reshape(-1, ECHUNK)], axis=1)
    epad_c = NCH_C * ECHUNK * 16 - E
    padg_c = N_C + jnp.remainder(jnp.arange(epad_c, dtype=i32), NPAD - N_C)
    dst2d = jnp.concatenate([ei0, padg_c, ei1, padg_c]).reshape(-1, ECHUNK)

    lpad = L_PAD - L
    lpz = jnp.zeros((lpad,), i32)
    ia = jnp.concatenate([edge_label_index[0], lpz])
    ib = jnp.concatenate([edge_label_index[1] + N_C, lpz])

    zeros = jnp.zeros((NPAD, H), jnp.float32)
    ones = jnp.ones((ECHUNK, H), jnp.float32)

    x_c = _encode(x_user, W_user_lin, b_user_lin, user_emb)
    x_j = _encode(x_item, W_job_lin, b_job_lin, job_emb)
    xcj = jnp.concatenate([x_c, x_j], axis=0)

    wl = (jnp.stack([Wl0_j2c, Wl0_c2j]), jnp.stack([Wl1_j2c, Wl1_c2j]),
          jnp.stack([Wl2_j2c, Wl2_c2j]))
    wr = (jnp.stack([Wr0_j2c, Wr0_c2j]), jnp.stack([Wr1_j2c, Wr1_c2j]),
          jnp.stack([Wr2_j2c, Wr2_c2j]))
    bb = (jnp.stack([b0_j2c, b0_c2j]), jnp.stack([b1_j2c, b1_c2j]),
          jnp.stack([b2_j2c, b2_c2j]))

    cnts = _sc_counts(dst2d, zeros, ones)
    for l in range(3):
        sums = _sc_agg(xcj, sd3, zeros)
        xcj = _layer(sums, cnts, xcj, wl[l], wr[l], bb[l], relu=(l == 0))

    pred = _sc_classify(xcj, ia, ib)
    return pred[:L]
